# baseline (speedup 1.0000x reference)
import functools

import jax
import jax.numpy as jnp
from jax import lax
from jax.experimental import pallas as pl
from jax.experimental.pallas import tpu as pltpu
from jax.experimental.pallas import tpu_sc as plsc

_NUM_SC_CORES = 2
_NUM_SC_SUBCORES = 16
_NW = _NUM_SC_CORES * _NUM_SC_SUBCORES
_CHUNK = 64


def _sc_scatter_rows(x, pos):
    n, d = x.shape
    per_w = n // _NW
    n_chunks = per_w // _CHUNK
    mesh = plsc.VectorSubcoreMesh(core_axis_name="c", subcore_axis_name="s")
    pos2 = pos.reshape(n // _CHUNK, _CHUNK)

    @functools.partial(
        pl.kernel,
        out_type=jax.ShapeDtypeStruct((n, d), jnp.float32),
        mesh=mesh,
        scratch_types=[
            pltpu.VMEM((n_chunks, _CHUNK), jnp.int32),
            pltpu.VMEM((_CHUNK, d), jnp.float32),
            pltpu.VMEM((_CHUNK, d), jnp.float32),
            pltpu.SemaphoreType.DMA,
            pltpu.SemaphoreType.DMA,
            pltpu.SemaphoreType.DMA,
            pltpu.SemaphoreType.DMA,
        ],
    )
    def scat(x_hbm, pos_hbm, xs_hbm, idx_v, rows0, rows1, sf0, sf1, ss0, ss1):
        wid = lax.axis_index("s") * _NUM_SC_CORES + lax.axis_index("c")
        base = wid * per_w
        pltpu.sync_copy(pos_hbm.at[pl.ds(wid * n_chunks, n_chunks)], idx_v)
        rows = (rows0, rows1)
        sf = (sf0, sf1)
        ss = (ss0, ss1)
        fetches = [None] * n_chunks
        scats = [None] * n_chunks
        for c in range(n_chunks):
            b = c % 2
            if c >= 2:
                scats[c - 2].wait()
            fetches[c] = pltpu.async_copy(
                x_hbm.at[pl.ds(base + c * _CHUNK, _CHUNK)], rows[b], sf[b]
            )
            fetches[c].wait()
            scats[c] = pltpu.async_copy(rows[b], xs_hbm.at[idx_v.at[c]], ss[b])
        for c in range(max(0, n_chunks - 2), n_chunks):
            scats[c].wait()

    return scat(x, pos2)


def _sc_gather_rows(y_sorted, pos):
    n, d = y_sorted.shape
    per_w = n // _NW
    n_chunks = per_w // _CHUNK
    mesh = plsc.VectorSubcoreMesh(core_axis_name="c", subcore_axis_name="s")
    pos2 = pos.reshape(n // _CHUNK, _CHUNK)

    @functools.partial(
        pl.kernel,
        out_type=jax.ShapeDtypeStruct((n, d), jnp.float32),
        mesh=mesh,
        scratch_types=[
            pltpu.VMEM((n_chunks, _CHUNK), jnp.int32),
            pltpu.VMEM((_CHUNK, d), jnp.float32),
            pltpu.VMEM((_CHUNK, d), jnp.float32),
            pltpu.SemaphoreType.DMA,
            pltpu.SemaphoreType.DMA,
            pltpu.SemaphoreType.DMA,
            pltpu.SemaphoreType.DMA,
        ],
    )
    def gat(ys_hbm, pos_hbm, out_hbm, idx_v, rows0, rows1, sg0, sg1, sw0, sw1):
        wid = lax.axis_index("s") * _NUM_SC_CORES + lax.axis_index("c")
        base = wid * per_w
        pltpu.sync_copy(pos_hbm.at[pl.ds(wid * n_chunks, n_chunks)], idx_v)
        rows = (rows0, rows1)
        sg = (sg0, sg1)
        sw = (sw0, sw1)
        gathers = [None] * n_chunks
        writes = [None] * n_chunks
        for c in range(n_chunks):
            b = c % 2
            if c >= 2:
                writes[c - 2].wait()
            gathers[c] = pltpu.async_copy(ys_hbm.at[idx_v.at[c]], rows[b], sg[b])
            gathers[c].wait()
            writes[c] = pltpu.async_copy(
                rows[b], out_hbm.at[pl.ds(base + c * _CHUNK, _CHUNK)], sw[b]
            )
        for c in range(max(0, n_chunks - 2), n_chunks):
            writes[c].wait()

    return gat(y_sorted, pos2)

_TILE_M = 256
_CB = 1024


def _row_from_col(col):
    k = col.shape[0]
    eye = (
        jax.lax.broadcasted_iota(jnp.int32, (k, k), 0)
        == jax.lax.broadcasted_iota(jnp.int32, (k, k), 1)
    ).astype(jnp.float32)
    return jnp.sum(eye * col, axis=0, keepdims=True)


def _build_meta(counts_row, offs_row, n, tile_m):
    e = counts_row.shape[1]
    t = n // tile_m
    g_max = t + e
    ends_row = offs_row + counts_row

    t_col = jax.lax.broadcasted_iota(jnp.int32, (t, e), 0).astype(jnp.float32)
    inter = (offs_row < (t_col + 1.0) * tile_m) & (ends_row > t_col * tile_m)
    intf = inter.astype(jnp.float32)

    units_col = jnp.sum(intf, axis=1, keepdims=True)
    units_row = _row_from_col(units_col)
    ut_incl = (
        jax.lax.broadcasted_iota(jnp.int32, (t, t), 0)
        <= jax.lax.broadcasted_iota(jnp.int32, (t, t), 1)
    ).astype(jnp.float32)
    c_incl_row = jnp.dot(units_row, ut_incl, preferred_element_type=jnp.float32)

    g_iota = jax.lax.broadcasted_iota(jnp.int32, (g_max, t), 0).astype(jnp.float32)
    before = (c_incl_row <= g_iota).astype(jnp.float32)
    g_t_col = jnp.sum(before, axis=1, keepdims=True)
    g_t_col = jnp.minimum(g_t_col, float(t - 1))
    c_excl_at = jnp.sum(before * units_row, axis=1, keepdims=True)
    g_col1 = jax.lax.broadcasted_iota(jnp.int32, (g_max, 1), 0).astype(jnp.float32)
    r_col = g_col1 - c_excl_at

    ut_incl_e = (
        jax.lax.broadcasted_iota(jnp.int32, (e, e), 0)
        <= jax.lax.broadcasted_iota(jnp.int32, (e, e), 1)
    ).astype(jnp.float32)
    prefix_incl = jnp.dot(intf, ut_incl_e, preferred_element_type=jnp.float32)
    sel_t = (
        jax.lax.broadcasted_iota(jnp.int32, (g_max, t), 1).astype(jnp.float32) == g_t_col
    ).astype(jnp.float32)
    p_ge = jnp.dot(sel_t, prefix_incl, preferred_element_type=jnp.float32)
    g_e_col = jnp.sum((p_ge <= r_col).astype(jnp.float32), axis=1, keepdims=True)
    g_e_col = jnp.minimum(g_e_col, float(e - 1))

    sel_e = (
        jax.lax.broadcasted_iota(jnp.int32, (g_max, e), 1).astype(jnp.float32) == g_e_col
    ).astype(jnp.float32)
    offs_at = jnp.sum(sel_e * offs_row, axis=1, keepdims=True)
    ends_at = jnp.sum(sel_e * ends_row, axis=1, keepdims=True)

    total_units = jnp.sum(units_row)
    valid = g_col1 < total_units
    g_start_col = jnp.where(valid, jnp.maximum(g_t_col * tile_m, offs_at), 0.0)
    g_end_col = jnp.where(
        valid, jnp.minimum((g_t_col + 1.0) * tile_m, ends_at), 0.0
    )

    pad = jnp.zeros((1, 128 - g_max), jnp.float32)
    rows = [
        jnp.concatenate([_row_from_col(c), pad], axis=1)
        for c in (g_t_col, g_e_col, g_start_col, g_end_col)
    ]
    rows.append(jnp.zeros((4, 128), jnp.float32))
    return jnp.concatenate(rows, axis=0).astype(jnp.int32)


_RC = 1024


def _routing2_body(split_ref, pos_ref, meta_ref, oh_ref, tri_ref, *, n, e, tile_m):
    nk = n // _RC

    counts_col = jnp.zeros((e, 1), jnp.float32)
    lane_e = jax.lax.broadcasted_iota(jnp.int32, (e, _RC), 0)
    for k in range(nk):
        sp = split_ref[pl.ds(k, 1), :]
        oht = (lane_e == sp).astype(jnp.float32)
        oh_ref[k] = oht
        counts_col = counts_col + jnp.sum(oht, axis=1, keepdims=True)

    tri_e = (
        jax.lax.broadcasted_iota(jnp.int32, (e, e), 1)
        < jax.lax.broadcasted_iota(jnp.int32, (e, e), 0)
    ).astype(jnp.float32)
    offs_col = jnp.dot(
        tri_e,
        counts_col,
        preferred_element_type=jnp.float32,
        precision=jax.lax.Precision.HIGHEST,
    )

    meta_ref[...] = _build_meta(
        _row_from_col(counts_col), _row_from_col(offs_col), n, tile_m
    )

    tri_ref[...] = (
        jax.lax.broadcasted_iota(jnp.int32, (_RC, _RC), 0)
        < jax.lax.broadcasted_iota(jnp.int32, (_RC, _RC), 1)
    ).astype(jnp.float32)

    base_col = jnp.zeros((e, 1), jnp.float32)
    for k in range(nk):
        oht = oh_ref[k]
        rank = jnp.dot(oht, tri_ref[...], preferred_element_type=jnp.float32)
        val = (offs_col + base_col + rank) * oht
        pos_ref[pl.ds(k, 1), :] = jnp.sum(val, axis=0, keepdims=True).astype(
            jnp.int32
        )
        base_col = base_col + jnp.sum(oht, axis=1, keepdims=True)


def _routing(split, n, e):
    nk = n // _RC
    pos2, meta = pl.pallas_call(
        functools.partial(_routing2_body, n=n, e=e, tile_m=_TILE_M),
        grid=(1,),
        in_specs=[pl.BlockSpec((nk, _RC), lambda i: (0, 0))],
        out_specs=[
            pl.BlockSpec((nk, _RC), lambda i: (0, 0)),
            pl.BlockSpec((8, 128), lambda i: (0, 0)),
        ],
        scratch_shapes=[
            pltpu.VMEM((nk, e, _RC), jnp.float32),
            pltpu.VMEM((_RC, _RC), jnp.float32),
        ],
        out_shape=[
            jax.ShapeDtypeStruct((nk, _RC), jnp.int32),
            jax.ShapeDtypeStruct((8, 128), jnp.int32),
        ],
    )(split.reshape(nk, _RC))
    return pos2.reshape(n), meta


def _gmm_body(meta, x_ref, w_ref, b_ref, o_ref, *, tile_m):
    g = pl.program_id(0)
    y = jnp.dot(
        x_ref[...].astype(jnp.bfloat16),
        w_ref[0].astype(jnp.bfloat16),
        preferred_element_type=jnp.float32,
    )
    y = y + b_ref[0]
    row = meta[0, g] * tile_m + jax.lax.broadcasted_iota(jnp.int32, (tile_m, 1), 0)
    mask = (row >= meta[2, g]) & (row < meta[3, g])
    o_ref[...] = jnp.where(mask, y, o_ref[...])


def _grouped_matmul(x_sorted, W, b3, meta, g_max):
    n, d = x_sorted.shape
    tile_m = _TILE_M
    grid_spec = pltpu.PrefetchScalarGridSpec(
        num_scalar_prefetch=1,
        grid=(g_max,),
        in_specs=[
            pl.BlockSpec((tile_m, d), lambda g, meta: (meta[0, g], 0)),
            pl.BlockSpec((1, d, d), lambda g, meta: (meta[1, g], 0, 0)),
            pl.BlockSpec((1, 1, d), lambda g, meta: (meta[1, g], 0, 0)),
        ],
        out_specs=pl.BlockSpec((tile_m, d), lambda g, meta: (meta[0, g], 0)),
    )
    return pl.pallas_call(
        functools.partial(_gmm_body, tile_m=tile_m),
        grid_spec=grid_spec,
        out_shape=jax.ShapeDtypeStruct((n, d), jnp.float32),
    )(meta, x_sorted, W, b3)


def kernel(x, split, W, b):
    n, d = x.shape
    e = W.shape[0]
    split = split.astype(jnp.int32)

    pos, meta = _routing(split, n, e)
    x_sorted = _sc_scatter_rows(x, pos)

    y_sorted = _grouped_matmul(
        x_sorted, W, b.reshape(e, 1, d), meta, n // _TILE_M + e
    )
    return _sc_gather_rows(y_sorted, pos)


# speedup vs baseline: 1.0245x; 1.0245x over previous
import functools

import jax
import jax.numpy as jnp
from jax import lax
from jax.experimental import pallas as pl
from jax.experimental.pallas import tpu as pltpu
from jax.experimental.pallas import tpu_sc as plsc

_NUM_SC_CORES = 2
_NUM_SC_SUBCORES = 16
_NW = _NUM_SC_CORES * _NUM_SC_SUBCORES
_CHUNK = 128


def _sc_scatter_rows(x, pos):
    n, d = x.shape
    per_w = n // _NW
    n_chunks = per_w // _CHUNK
    mesh = plsc.VectorSubcoreMesh(core_axis_name="c", subcore_axis_name="s")

    @functools.partial(
        pl.kernel,
        out_type=jax.ShapeDtypeStruct((n, d), jnp.float32),
        mesh=mesh,
        scratch_types=[
            pltpu.VMEM((_CHUNK,), jnp.int32),
            pltpu.VMEM((_CHUNK, d), jnp.float32),
            pltpu.SemaphoreType.DMA,
        ],
    )
    def scat(x_hbm, pos_hbm, xs_hbm, idx_v, rows_v, sem):
        wid = lax.axis_index("s") * _NUM_SC_CORES + lax.axis_index("c")
        base = wid * per_w
        for c in range(n_chunks):
            off = base + c * _CHUNK
            pltpu.sync_copy(pos_hbm.at[pl.ds(off, _CHUNK)], idx_v)
            pltpu.sync_copy(x_hbm.at[pl.ds(off, _CHUNK)], rows_v)
            pltpu.async_copy(rows_v, xs_hbm.at[idx_v], sem).wait()

    return scat(x, pos)


def _sc_gather_rows(y_sorted, pos):
    n, d = y_sorted.shape
    per_w = n // _NW
    n_chunks = per_w // _CHUNK
    mesh = plsc.VectorSubcoreMesh(core_axis_name="c", subcore_axis_name="s")

    @functools.partial(
        pl.kernel,
        out_type=jax.ShapeDtypeStruct((n, d), jnp.float32),
        mesh=mesh,
        scratch_types=[
            pltpu.VMEM((_CHUNK,), jnp.int32),
            pltpu.VMEM((_CHUNK, d), jnp.float32),
            pltpu.SemaphoreType.DMA,
        ],
    )
    def gat(ys_hbm, pos_hbm, out_hbm, idx_v, rows_v, sem):
        wid = lax.axis_index("s") * _NUM_SC_CORES + lax.axis_index("c")
        base = wid * per_w
        for c in range(n_chunks):
            off = base + c * _CHUNK
            pltpu.sync_copy(pos_hbm.at[pl.ds(off, _CHUNK)], idx_v)
            pltpu.async_copy(ys_hbm.at[idx_v], rows_v, sem).wait()
            pltpu.sync_copy(rows_v, out_hbm.at[pl.ds(off, _CHUNK)])

    return gat(y_sorted, pos)

_TILE_M = 256
_CB = 1024


def _row_from_col(col):
    k = col.shape[0]
    eye = (
        jax.lax.broadcasted_iota(jnp.int32, (k, k), 0)
        == jax.lax.broadcasted_iota(jnp.int32, (k, k), 1)
    ).astype(jnp.float32)
    return jnp.sum(eye * col, axis=0, keepdims=True)


def _build_meta(counts_row, offs_row, n, tile_m):
    e = counts_row.shape[1]
    t = n // tile_m
    g_max = t + e
    ends_row = offs_row + counts_row

    t_col = jax.lax.broadcasted_iota(jnp.int32, (t, e), 0).astype(jnp.float32)
    inter = (offs_row < (t_col + 1.0) * tile_m) & (ends_row > t_col * tile_m)
    intf = inter.astype(jnp.float32)

    units_col = jnp.sum(intf, axis=1, keepdims=True)
    units_row = _row_from_col(units_col)
    ut_incl = (
        jax.lax.broadcasted_iota(jnp.int32, (t, t), 0)
        <= jax.lax.broadcasted_iota(jnp.int32, (t, t), 1)
    ).astype(jnp.float32)
    c_incl_row = jnp.dot(units_row, ut_incl, preferred_element_type=jnp.float32)

    g_iota = jax.lax.broadcasted_iota(jnp.int32, (g_max, t), 0).astype(jnp.float32)
    before = (c_incl_row <= g_iota).astype(jnp.float32)
    g_t_col = jnp.sum(before, axis=1, keepdims=True)
    g_t_col = jnp.minimum(g_t_col, float(t - 1))
    c_excl_at = jnp.sum(before * units_row, axis=1, keepdims=True)
    g_col1 = jax.lax.broadcasted_iota(jnp.int32, (g_max, 1), 0).astype(jnp.float32)
    r_col = g_col1 - c_excl_at

    ut_incl_e = (
        jax.lax.broadcasted_iota(jnp.int32, (e, e), 0)
        <= jax.lax.broadcasted_iota(jnp.int32, (e, e), 1)
    ).astype(jnp.float32)
    prefix_incl = jnp.dot(intf, ut_incl_e, preferred_element_type=jnp.float32)
    sel_t = (
        jax.lax.broadcasted_iota(jnp.int32, (g_max, t), 1).astype(jnp.float32) == g_t_col
    ).astype(jnp.float32)
    p_ge = jnp.dot(sel_t, prefix_incl, preferred_element_type=jnp.float32)
    g_e_col = jnp.sum((p_ge <= r_col).astype(jnp.float32), axis=1, keepdims=True)
    g_e_col = jnp.minimum(g_e_col, float(e - 1))

    sel_e = (
        jax.lax.broadcasted_iota(jnp.int32, (g_max, e), 1).astype(jnp.float32) == g_e_col
    ).astype(jnp.float32)
    offs_at = jnp.sum(sel_e * offs_row, axis=1, keepdims=True)
    ends_at = jnp.sum(sel_e * ends_row, axis=1, keepdims=True)

    total_units = jnp.sum(units_row)
    valid = g_col1 < total_units
    g_start_col = jnp.where(valid, jnp.maximum(g_t_col * tile_m, offs_at), 0.0)
    g_end_col = jnp.where(
        valid, jnp.minimum((g_t_col + 1.0) * tile_m, ends_at), 0.0
    )

    pad = jnp.zeros((1, 128 - g_max), jnp.float32)
    rows = [
        jnp.concatenate([_row_from_col(c), pad], axis=1)
        for c in (g_t_col, g_e_col, g_start_col, g_end_col)
    ]
    rows.append(jnp.zeros((4, 128), jnp.float32))
    return jnp.concatenate(rows, axis=0).astype(jnp.int32)


_RC = 1024


def _routing2_body(split_ref, pos_ref, meta_ref, oh_ref, tri_ref, *, n, e, tile_m):
    nk = n // _RC

    counts_col = jnp.zeros((e, 1), jnp.float32)
    lane_e = jax.lax.broadcasted_iota(jnp.int32, (e, _RC), 0)
    for k in range(nk):
        sp = split_ref[pl.ds(k, 1), :]
        oht = (lane_e == sp).astype(jnp.float32)
        oh_ref[k] = oht
        counts_col = counts_col + jnp.sum(oht, axis=1, keepdims=True)

    tri_e = (
        jax.lax.broadcasted_iota(jnp.int32, (e, e), 1)
        < jax.lax.broadcasted_iota(jnp.int32, (e, e), 0)
    ).astype(jnp.float32)
    offs_col = jnp.dot(
        tri_e,
        counts_col,
        preferred_element_type=jnp.float32,
        precision=jax.lax.Precision.HIGHEST,
    )

    meta_ref[...] = _build_meta(
        _row_from_col(counts_col), _row_from_col(offs_col), n, tile_m
    )

    tri_ref[...] = (
        jax.lax.broadcasted_iota(jnp.int32, (_RC, _RC), 0)
        < jax.lax.broadcasted_iota(jnp.int32, (_RC, _RC), 1)
    ).astype(jnp.float32)

    base_col = jnp.zeros((e, 1), jnp.float32)
    for k in range(nk):
        oht = oh_ref[k]
        rank = jnp.dot(oht, tri_ref[...], preferred_element_type=jnp.float32)
        val = (offs_col + base_col + rank) * oht
        pos_ref[pl.ds(k, 1), :] = jnp.sum(val, axis=0, keepdims=True).astype(
            jnp.int32
        )
        base_col = base_col + jnp.sum(oht, axis=1, keepdims=True)


def _routing(split, n, e):
    nk = n // _RC
    pos2, meta = pl.pallas_call(
        functools.partial(_routing2_body, n=n, e=e, tile_m=_TILE_M),
        grid=(1,),
        in_specs=[pl.BlockSpec((nk, _RC), lambda i: (0, 0))],
        out_specs=[
            pl.BlockSpec((nk, _RC), lambda i: (0, 0)),
            pl.BlockSpec((8, 128), lambda i: (0, 0)),
        ],
        scratch_shapes=[
            pltpu.VMEM((nk, e, _RC), jnp.float32),
            pltpu.VMEM((_RC, _RC), jnp.float32),
        ],
        out_shape=[
            jax.ShapeDtypeStruct((nk, _RC), jnp.int32),
            jax.ShapeDtypeStruct((8, 128), jnp.int32),
        ],
    )(split.reshape(nk, _RC))
    return pos2.reshape(n), meta


def _gmm_body(meta, x_ref, w_ref, b_ref, o_ref, *, tile_m):
    g = pl.program_id(0)
    y = jnp.dot(
        x_ref[...].astype(jnp.bfloat16),
        w_ref[0].astype(jnp.bfloat16),
        preferred_element_type=jnp.float32,
    )
    y = y + b_ref[0]
    row = meta[0, g] * tile_m + jax.lax.broadcasted_iota(jnp.int32, (tile_m, 1), 0)
    mask = (row >= meta[2, g]) & (row < meta[3, g])
    o_ref[...] = jnp.where(mask, y, o_ref[...])


def _grouped_matmul(x_sorted, W, b3, meta, g_max):
    n, d = x_sorted.shape
    tile_m = _TILE_M
    grid_spec = pltpu.PrefetchScalarGridSpec(
        num_scalar_prefetch=1,
        grid=(g_max,),
        in_specs=[
            pl.BlockSpec((tile_m, d), lambda g, meta: (meta[0, g], 0)),
            pl.BlockSpec((1, d, d), lambda g, meta: (meta[1, g], 0, 0)),
            pl.BlockSpec((1, 1, d), lambda g, meta: (meta[1, g], 0, 0)),
        ],
        out_specs=pl.BlockSpec((tile_m, d), lambda g, meta: (meta[0, g], 0)),
    )
    return pl.pallas_call(
        functools.partial(_gmm_body, tile_m=tile_m),
        grid_spec=grid_spec,
        out_shape=jax.ShapeDtypeStruct((n, d), jnp.float32),
    )(meta, x_sorted, W, b3)


def kernel(x, split, W, b):
    n, d = x.shape
    e = W.shape[0]
    split = split.astype(jnp.int32)

    pos, meta = _routing(split, n, e)
    x_sorted = _sc_scatter_rows(x, pos)

    y_sorted = _grouped_matmul(
        x_sorted, W, b.reshape(e, 1, d), meta, n // _TILE_M + e
    )
    return _sc_gather_rows(y_sorted, pos)
